# baseline (speedup 1.0000x reference)
import functools

import jax
import jax.numpy as jnp
from jax import lax
from jax.experimental import pallas as pl
from jax.experimental.pallas import tpu as pltpu
from jax.experimental.pallas import tpu_sc as plsc

N_NODES = 10000
N_GRAPHS = 64
D = 128

NPAD = 10240
N_TILES = 32
EC = 128
EH = EC // 2
CPT = 80
PHW = 16
NPH = CPT // PHW
EPT = EC * CPT
EPAD = N_TILES * EPT
ROWS_PER_TILE = NPAD // 16

_R = 2560


def _sc_mesh():
    return plsc.VectorSubcoreMesh(core_axis_name="c", subcore_axis_name="s")


@functools.partial(
    pl.kernel,
    out_type=jax.ShapeDtypeStruct((N_TILES, NPAD), jnp.float32),
    mesh=_sc_mesh(),
    compiler_params=pltpu.CompilerParams(needs_layout_passes=False),
    scratch_types=[
        pltpu.VMEM((CPT, EC), jnp.int32),
        pltpu.VMEM((NPAD,), jnp.float32),
    ],
)
def _sc_degree(edges_hbm, out_hbm, dst_v, hist_v):
    cid = lax.axis_index("c")
    sid = lax.axis_index("s")
    wid = sid * 2 + cid

    def zero_body(i, _):
        hist_v[pl.ds(i * 16, 16)] = jnp.zeros((16,), jnp.float32)
        return 0

    lax.fori_loop(0, NPAD // 16, zero_body, 0)

    pltpu.sync_copy(edges_hbm.at[1, pl.ds(wid * CPT, CPT)], dst_v)

    ones = jnp.full((16,), 1.0, jnp.float32)

    def body(r, _):
        for j in range(EC // 16):
            idx = dst_v[r, pl.ds(j * 16, 16)]
            plsc.addupdate_scatter(hist_v, [idx], ones)
        return 0

    lax.fori_loop(0, CPT, body, 0)
    pltpu.sync_copy(hist_v, out_hbm.at[wid])


@functools.partial(
    pl.kernel,
    out_type=jax.ShapeDtypeStruct((2, NPAD, D), jnp.float32),
    mesh=_sc_mesh(),
    compiler_params=pltpu.CompilerParams(needs_layout_passes=False),
    scratch_types=[
        pltpu.VMEM((2, 2, PHW, EC), jnp.int32),
        pltpu.VMEM((2, EC, D), jnp.float32),
        pltpu.VMEM_SHARED((NPAD, D), jnp.float32),
        pltpu.SemaphoreType.DMA,
        pltpu.SemaphoreType.DMA,
        pltpu.SemaphoreType.DMA,
    ],
)
def _sc_edge_pass(table_hbm, edges_hbm, out_hbm, idx_v, rows_v, acc_sh,
                  sem_a, sem_b, isem):
    cid = lax.axis_index("c")
    sid = lax.axis_index("s")
    wid = sid * 2 + cid

    def zero_body(i, _):
        for j in range(D // 16):
            rows_v[0, i, pl.ds(j * 16, 16)] = jnp.zeros((16,), jnp.float32)
        return 0

    lax.fori_loop(0, EC, zero_body, 0)
    row0 = sid * ROWS_PER_TILE
    for t in range(ROWS_PER_TILE // EC):
        pltpu.sync_copy(rows_v.at[0], acc_sh.at[pl.ds(row0 + t * EC, EC)])

    base = wid * CPT
    pltpu.sync_copy(edges_hbm.at[:, pl.ds(base, PHW)], idx_v.at[0])
    ipf = pltpu.async_copy(edges_hbm.at[:, pl.ds(base + PHW, PHW)],
                           idx_v.at[1], isem)

    plsc.subcore_barrier()

    def gather_pair(pp, cc, buf):
        eq = EC // 4
        a = pltpu.async_copy(table_hbm.at[idx_v.at[pp, 0, cc, pl.ds(0, eq)]],
                             rows_v.at[buf, pl.ds(0, eq)], sem_a)
        b = pltpu.async_copy(table_hbm.at[idx_v.at[pp, 0, cc, pl.ds(eq, eq)]],
                             rows_v.at[buf, pl.ds(eq, eq)], sem_b)
        c = pltpu.async_copy(
            table_hbm.at[idx_v.at[pp, 0, cc, pl.ds(2 * eq, eq)]],
            rows_v.at[buf, pl.ds(2 * eq, eq)], sem_a)
        d = pltpu.async_copy(
            table_hbm.at[idx_v.at[pp, 0, cc, pl.ds(3 * eq, eq)]],
            rows_v.at[buf, pl.ds(3 * eq, eq)], sem_b)
        return a, b, c, d

    cp = gather_pair(0, 0, 0)
    for g in range(CPT):
        p, c = divmod(g, PHW)
        for d in cp:
            d.wait()
        if g + 1 < CPT:
            p1, c1 = divmod(g + 1, PHW)
            if c1 == 0:
                ipf.wait()
            cp = gather_pair(p1 % 2, c1, (g + 1) % 2)
        pltpu.sync_copy(rows_v.at[g % 2], acc_sh.at[idx_v.at[p % 2, 1, c]],
                        add=True)
        if c == PHW - 1 and p + 2 < NPH:
            ipf = pltpu.async_copy(
                edges_hbm.at[:, pl.ds(base + (p + 2) * PHW, PHW)],
                idx_v.at[p % 2], isem)

    plsc.subcore_barrier()
    pltpu.sync_copy(acc_sh.at[pl.ds(row0, ROWS_PER_TILE)],
                    out_hbm.at[cid, pl.ds(row0, ROWS_PER_TILE)])


def _tc_layer1(x_pad, W1, degT):
    def body(x_ref, w_ref, deg_ref, t_ref, dinv_ref):
        deg = jnp.sum(deg_ref[...], axis=1, keepdims=True) + 1.0
        dinv = lax.rsqrt(jnp.maximum(deg, 1.0))
        t_ref[...] = jnp.dot(x_ref[...], w_ref[...],
                             preferred_element_type=jnp.float32) * dinv
        dinv_ref[...] = dinv

    return pl.pallas_call(
        body,
        grid=(NPAD // _R,),
        in_specs=[
            pl.BlockSpec((_R, D), lambda i: (i, 0)),
            pl.BlockSpec((D, D), lambda i: (0, 0)),
            pl.BlockSpec((_R, N_TILES), lambda i: (i, 0)),
        ],
        out_specs=[
            pl.BlockSpec((_R, D), lambda i: (i, 0)),
            pl.BlockSpec((_R, 1), lambda i: (i, 0)),
        ],
        out_shape=[
            jax.ShapeDtypeStruct((NPAD, D), jnp.float32),
            jax.ShapeDtypeStruct((NPAD, 1), jnp.float32),
        ],
    )(x_pad, W1, degT)


def _tc_mid(s1, table1, dinv, b1r, W2):
    def body(s_ref, t_ref, dinv_ref, b_ref, w_ref, o_ref):
        h = jax.nn.relu((s_ref[0] + s_ref[1] + t_ref[...]) * dinv_ref[...]
                        + b_ref[...])
        o_ref[...] = jnp.dot(h, w_ref[...],
                             preferred_element_type=jnp.float32) * dinv_ref[...]

    return pl.pallas_call(
        body,
        grid=(NPAD // _R,),
        in_specs=[
            pl.BlockSpec((2, _R, D), lambda i: (0, i, 0)),
            pl.BlockSpec((_R, D), lambda i: (i, 0)),
            pl.BlockSpec((_R, 1), lambda i: (i, 0)),
            pl.BlockSpec((1, D), lambda i: (0, 0)),
            pl.BlockSpec((D, D), lambda i: (0, 0)),
        ],
        out_specs=pl.BlockSpec((_R, D), lambda i: (i, 0)),
        out_shape=jax.ShapeDtypeStruct((NPAD, D), jnp.float32),
    )(s1, table1, dinv, b1r, W2)


def _tc_final(s2, table2, dinv, b2r, batch_row, hlr, std,
              Wf1a, w_hlr, w_std, bf1r, Wf2, bf2r):
    def body(s_ref, t_ref, dinv_ref, b_ref, batch_ref, hlr_ref, std_ref,
             wa_ref, wh_ref, ws_ref, bf1_ref, wf2_ref, bf2_ref,
             o_ref, sums, counts):
        i = pl.program_id(0)
        h2 = jax.nn.relu((s_ref[0] + s_ref[1] + t_ref[...]) * dinv_ref[...]
                         + b_ref[...])
        gids = lax.broadcasted_iota(jnp.int32, (N_GRAPHS, _R), 0)
        ind = (batch_ref[...] == gids).astype(jnp.float32)
        psum = jnp.dot(ind, h2, preferred_element_type=jnp.float32)
        pcnt = jnp.sum(ind, axis=1, keepdims=True)

        @pl.when(i == 0)
        def _():
            sums[...] = jnp.zeros_like(sums)
            counts[...] = jnp.zeros_like(counts)

        sums[...] += psum
        counts[...] += pcnt

        @pl.when(i == pl.num_programs(0) - 1)
        def _():
            pooled = sums[...] / jnp.maximum(counts[...], 1.0)
            hfc = jax.nn.relu(
                jnp.dot(pooled, wa_ref[...], preferred_element_type=jnp.float32)
                + hlr_ref[...] * wh_ref[...]
                + std_ref[...] * ws_ref[...]
                + bf1_ref[...])
            o_ref[...] = (jnp.dot(hfc, wf2_ref[...],
                                  preferred_element_type=jnp.float32)
                          + bf2_ref[...])

    return pl.pallas_call(
        body,
        grid=(NPAD // _R,),
        in_specs=[
            pl.BlockSpec((2, _R, D), lambda i: (0, i, 0)),
            pl.BlockSpec((_R, D), lambda i: (i, 0)),
            pl.BlockSpec((_R, 1), lambda i: (i, 0)),
            pl.BlockSpec((1, D), lambda i: (0, 0)),
            pl.BlockSpec((1, _R), lambda i: (0, i)),
            pl.BlockSpec((N_GRAPHS, 1), lambda i: (0, 0)),
            pl.BlockSpec((N_GRAPHS, 1), lambda i: (0, 0)),
            pl.BlockSpec((D, D), lambda i: (0, 0)),
            pl.BlockSpec((1, D), lambda i: (0, 0)),
            pl.BlockSpec((1, D), lambda i: (0, 0)),
            pl.BlockSpec((1, D), lambda i: (0, 0)),
            pl.BlockSpec((D, D), lambda i: (0, 0)),
            pl.BlockSpec((1, D), lambda i: (0, 0)),
        ],
        out_specs=pl.BlockSpec((N_GRAPHS, D), lambda i: (0, 0)),
        out_shape=jax.ShapeDtypeStruct((N_GRAPHS, D), jnp.float32),
        scratch_shapes=[
            pltpu.VMEM((N_GRAPHS, D), jnp.float32),
            pltpu.VMEM((N_GRAPHS, 1), jnp.float32),
        ],
    )(s2, table2, dinv, b2r, batch_row, hlr, std,
      Wf1a, w_hlr, w_std, bf1r, Wf2, bf2r)


def kernel(x, edge_index, batch, hlr, std, W1, b1, W2, b2, Wf1, bf1, Wf2, bf2):
    n_edges = edge_index.shape[1]
    n_extra = EPAD - n_edges
    pad_idx = N_NODES + (jnp.arange(n_extra, dtype=jnp.int32) % (NPAD - N_NODES))
    pad2 = jnp.broadcast_to(pad_idx, (2, n_extra))
    edges3d = jnp.concatenate([edge_index, pad2], axis=1).reshape(2, -1, EC)

    x_pad = jnp.pad(x, ((0, NPAD - N_NODES), (0, 0)))
    batch_row = jnp.pad(batch, (0, NPAD - N_NODES),
                        constant_values=N_GRAPHS).reshape(1, NPAD)

    b1r = b1.reshape(1, D)
    b2r = b2.reshape(1, D)
    bf1r = bf1.reshape(1, D)
    bf2r = bf2.reshape(1, D)
    Wf1a = Wf1[:D]
    w_hlr = Wf1[D:D + 1]
    w_std = Wf1[D + 1:D + 2]

    deg_parts = _sc_degree(edges3d)
    degT = jnp.transpose(deg_parts)

    table1, dinv = _tc_layer1(x_pad, W1, degT)
    s1 = _sc_edge_pass(table1, edges3d)
    table2 = _tc_mid(s1, table1, dinv, b1r, W2)
    s2 = _sc_edge_pass(table2, edges3d)
    return _tc_final(s2, table2, dinv, b2r, batch_row, hlr, std,
                     Wf1a, w_hlr, w_std, bf1r, Wf2, bf2r)


# speedup vs baseline: 1.0059x; 1.0059x over previous
import functools

import jax
import jax.numpy as jnp
from jax import lax
from jax.experimental import pallas as pl
from jax.experimental.pallas import tpu as pltpu
from jax.experimental.pallas import tpu_sc as plsc

N_NODES = 10000
N_GRAPHS = 64
D = 128

NPAD = 10240
N_TILES = 32
EC = 128
EH = EC // 2
CPT = 80
PHW = 16
NPH = CPT // PHW
EPT = EC * CPT
EPAD = N_TILES * EPT
ROWS_PER_TILE = NPAD // 16

_R = 2560


def _sc_mesh():
    return plsc.VectorSubcoreMesh(core_axis_name="c", subcore_axis_name="s")


@functools.partial(
    pl.kernel,
    out_type=jax.ShapeDtypeStruct((N_TILES, NPAD), jnp.float32),
    mesh=_sc_mesh(),
    compiler_params=pltpu.CompilerParams(needs_layout_passes=False),
    scratch_types=[
        pltpu.VMEM((CPT, EC), jnp.int32),
        pltpu.VMEM((NPAD,), jnp.float32),
    ],
)
def _sc_degree(edges_hbm, out_hbm, dst_v, hist_v):
    cid = lax.axis_index("c")
    sid = lax.axis_index("s")
    wid = sid * 2 + cid

    def zero_body(i, _):
        hist_v[pl.ds(i * 16, 16)] = jnp.zeros((16,), jnp.float32)
        return 0

    lax.fori_loop(0, NPAD // 16, zero_body, 0)

    pltpu.sync_copy(edges_hbm.at[1, pl.ds(wid * CPT, CPT)], dst_v)

    ones = jnp.full((16,), 1.0, jnp.float32)

    def body(r, _):
        for j in range(EC // 16):
            idx = dst_v[r, pl.ds(j * 16, 16)]
            plsc.addupdate_scatter(hist_v, [idx], ones)
        return 0

    lax.fori_loop(0, CPT, body, 0)
    pltpu.sync_copy(hist_v, out_hbm.at[wid])


@functools.partial(
    pl.kernel,
    out_type=jax.ShapeDtypeStruct((2, NPAD, D), jnp.float32),
    mesh=_sc_mesh(),
    compiler_params=pltpu.CompilerParams(needs_layout_passes=False),
    scratch_types=[
        pltpu.VMEM((2, 2, PHW, EC), jnp.int32),
        pltpu.VMEM((2, EC, D), jnp.float32),
        pltpu.VMEM_SHARED((NPAD, D), jnp.float32),
        pltpu.SemaphoreType.DMA,
        pltpu.SemaphoreType.DMA,
        pltpu.SemaphoreType.DMA,
    ],
)
def _sc_edge_pass(table_hbm, edges_hbm, out_hbm, idx_v, rows_v, acc_sh,
                  sem_a, sem_b, isem):
    cid = lax.axis_index("c")
    sid = lax.axis_index("s")
    wid = sid * 2 + cid

    def zero_body(i, _):
        for j in range(D // 16):
            rows_v[0, i, pl.ds(j * 16, 16)] = jnp.zeros((16,), jnp.float32)
        return 0

    lax.fori_loop(0, EC, zero_body, 0)
    row0 = sid * ROWS_PER_TILE
    for t in range(ROWS_PER_TILE // EC):
        pltpu.sync_copy(rows_v.at[0], acc_sh.at[pl.ds(row0 + t * EC, EC)])

    base = wid * CPT
    pltpu.sync_copy(edges_hbm.at[:, pl.ds(base, PHW)], idx_v.at[0])
    ipf = pltpu.async_copy(edges_hbm.at[:, pl.ds(base + PHW, PHW)],
                           idx_v.at[1], isem)

    plsc.subcore_barrier()

    def gather_pair(pp, cc, buf):
        a = pltpu.async_copy(table_hbm.at[idx_v.at[pp, 0, cc, pl.ds(0, EH)]],
                             rows_v.at[buf, pl.ds(0, EH)], sem_a)
        b = pltpu.async_copy(table_hbm.at[idx_v.at[pp, 0, cc, pl.ds(EH, EH)]],
                             rows_v.at[buf, pl.ds(EH, EH)], sem_b)
        return a, b

    cp = gather_pair(0, 0, 0)
    for g in range(CPT):
        p, c = divmod(g, PHW)
        for d in cp:
            d.wait()
        if g + 1 < CPT:
            p1, c1 = divmod(g + 1, PHW)
            if c1 == 0:
                ipf.wait()
            cp = gather_pair(p1 % 2, c1, (g + 1) % 2)
        pltpu.sync_copy(rows_v.at[g % 2], acc_sh.at[idx_v.at[p % 2, 1, c]],
                        add=True)
        if c == PHW - 1 and p + 2 < NPH:
            ipf = pltpu.async_copy(
                edges_hbm.at[:, pl.ds(base + (p + 2) * PHW, PHW)],
                idx_v.at[p % 2], isem)

    plsc.subcore_barrier()
    pltpu.sync_copy(acc_sh.at[pl.ds(row0, ROWS_PER_TILE)],
                    out_hbm.at[cid, pl.ds(row0, ROWS_PER_TILE)])


def _tc_layer1(x_pad, W1, degT):
    def body(x_ref, w_ref, deg_ref, t_ref, dinv_ref):
        deg = jnp.sum(deg_ref[...], axis=1, keepdims=True) + 1.0
        dinv = lax.rsqrt(jnp.maximum(deg, 1.0))
        t_ref[...] = jnp.dot(x_ref[...], w_ref[...],
                             preferred_element_type=jnp.float32) * dinv
        dinv_ref[...] = dinv

    return pl.pallas_call(
        body,
        grid=(NPAD // _R,),
        in_specs=[
            pl.BlockSpec((_R, D), lambda i: (i, 0)),
            pl.BlockSpec((D, D), lambda i: (0, 0)),
            pl.BlockSpec((_R, N_TILES), lambda i: (i, 0)),
        ],
        out_specs=[
            pl.BlockSpec((_R, D), lambda i: (i, 0)),
            pl.BlockSpec((_R, 1), lambda i: (i, 0)),
        ],
        out_shape=[
            jax.ShapeDtypeStruct((NPAD, D), jnp.float32),
            jax.ShapeDtypeStruct((NPAD, 1), jnp.float32),
        ],
    )(x_pad, W1, degT)


def _tc_mid(s1, table1, dinv, b1r, W2):
    def body(s_ref, t_ref, dinv_ref, b_ref, w_ref, o_ref):
        h = jax.nn.relu((s_ref[0] + s_ref[1] + t_ref[...]) * dinv_ref[...]
                        + b_ref[...])
        o_ref[...] = jnp.dot(h, w_ref[...],
                             preferred_element_type=jnp.float32) * dinv_ref[...]

    return pl.pallas_call(
        body,
        grid=(NPAD // _R,),
        in_specs=[
            pl.BlockSpec((2, _R, D), lambda i: (0, i, 0)),
            pl.BlockSpec((_R, D), lambda i: (i, 0)),
            pl.BlockSpec((_R, 1), lambda i: (i, 0)),
            pl.BlockSpec((1, D), lambda i: (0, 0)),
            pl.BlockSpec((D, D), lambda i: (0, 0)),
        ],
        out_specs=pl.BlockSpec((_R, D), lambda i: (i, 0)),
        out_shape=jax.ShapeDtypeStruct((NPAD, D), jnp.float32),
    )(s1, table1, dinv, b1r, W2)


def _tc_final(s2, table2, dinv, b2r, batch_row, hlr, std,
              Wf1a, w_hlr, w_std, bf1r, Wf2, bf2r):
    def body(s_ref, t_ref, dinv_ref, b_ref, batch_ref, hlr_ref, std_ref,
             wa_ref, wh_ref, ws_ref, bf1_ref, wf2_ref, bf2_ref,
             o_ref, sums, counts):
        i = pl.program_id(0)
        h2 = jax.nn.relu((s_ref[0] + s_ref[1] + t_ref[...]) * dinv_ref[...]
                         + b_ref[...])
        gids = lax.broadcasted_iota(jnp.int32, (N_GRAPHS, _R), 0)
        ind = (batch_ref[...] == gids).astype(jnp.float32)
        psum = jnp.dot(ind, h2, preferred_element_type=jnp.float32)
        pcnt = jnp.sum(ind, axis=1, keepdims=True)

        @pl.when(i == 0)
        def _():
            sums[...] = jnp.zeros_like(sums)
            counts[...] = jnp.zeros_like(counts)

        sums[...] += psum
        counts[...] += pcnt

        @pl.when(i == pl.num_programs(0) - 1)
        def _():
            pooled = sums[...] / jnp.maximum(counts[...], 1.0)
            hfc = jax.nn.relu(
                jnp.dot(pooled, wa_ref[...], preferred_element_type=jnp.float32)
                + hlr_ref[...] * wh_ref[...]
                + std_ref[...] * ws_ref[...]
                + bf1_ref[...])
            o_ref[...] = (jnp.dot(hfc, wf2_ref[...],
                                  preferred_element_type=jnp.float32)
                          + bf2_ref[...])

    return pl.pallas_call(
        body,
        grid=(NPAD // _R,),
        in_specs=[
            pl.BlockSpec((2, _R, D), lambda i: (0, i, 0)),
            pl.BlockSpec((_R, D), lambda i: (i, 0)),
            pl.BlockSpec((_R, 1), lambda i: (i, 0)),
            pl.BlockSpec((1, D), lambda i: (0, 0)),
            pl.BlockSpec((1, _R), lambda i: (0, i)),
            pl.BlockSpec((N_GRAPHS, 1), lambda i: (0, 0)),
            pl.BlockSpec((N_GRAPHS, 1), lambda i: (0, 0)),
            pl.BlockSpec((D, D), lambda i: (0, 0)),
            pl.BlockSpec((1, D), lambda i: (0, 0)),
            pl.BlockSpec((1, D), lambda i: (0, 0)),
            pl.BlockSpec((1, D), lambda i: (0, 0)),
            pl.BlockSpec((D, D), lambda i: (0, 0)),
            pl.BlockSpec((1, D), lambda i: (0, 0)),
        ],
        out_specs=pl.BlockSpec((N_GRAPHS, D), lambda i: (0, 0)),
        out_shape=jax.ShapeDtypeStruct((N_GRAPHS, D), jnp.float32),
        scratch_shapes=[
            pltpu.VMEM((N_GRAPHS, D), jnp.float32),
            pltpu.VMEM((N_GRAPHS, 1), jnp.float32),
        ],
    )(s2, table2, dinv, b2r, batch_row, hlr, std,
      Wf1a, w_hlr, w_std, bf1r, Wf2, bf2r)


def kernel(x, edge_index, batch, hlr, std, W1, b1, W2, b2, Wf1, bf1, Wf2, bf2):
    n_edges = edge_index.shape[1]
    n_extra = EPAD - n_edges
    pad_idx = N_NODES + (jnp.arange(n_extra, dtype=jnp.int32) % (NPAD - N_NODES))
    pad2 = jnp.broadcast_to(pad_idx, (2, n_extra))
    edges3d = jnp.concatenate([edge_index, pad2], axis=1).reshape(2, -1, EC)

    x_pad = jnp.pad(x, ((0, NPAD - N_NODES), (0, 0)))
    batch_row = jnp.pad(batch, (0, NPAD - N_NODES),
                        constant_values=N_GRAPHS).reshape(1, NPAD)

    b1r = b1.reshape(1, D)
    b2r = b2.reshape(1, D)
    bf1r = bf1.reshape(1, D)
    bf2r = bf2.reshape(1, D)
    Wf1a = Wf1[:D]
    w_hlr = Wf1[D:D + 1]
    w_std = Wf1[D + 1:D + 2]

    deg_parts = _sc_degree(edges3d)
    degT = jnp.transpose(deg_parts)

    table1, dinv = _tc_layer1(x_pad, W1, degT)
    s1 = _sc_edge_pass(table1, edges3d)
    table2 = _tc_mid(s1, table1, dinv, b1r, W2)
    s2 = _sc_edge_pass(table2, edges3d)
    return _tc_final(s2, table2, dinv, b2r, batch_row, hlr, std,
                     Wf1a, w_hlr, w_std, bf1r, Wf2, bf2r)


# speedup vs baseline: 1.0108x; 1.0049x over previous
import functools

import jax
import jax.numpy as jnp
from jax import lax
from jax.experimental import pallas as pl
from jax.experimental.pallas import tpu as pltpu
from jax.experimental.pallas import tpu_sc as plsc

N_NODES = 10000
N_GRAPHS = 64
D = 128

NPAD = 10240
N_TILES = 32
EC = 128
EH = EC // 2
CPT = 80
PHW = 16
NPH = CPT // PHW
EPT = EC * CPT
EPAD = N_TILES * EPT
ROWS_PER_TILE = NPAD // 16

_R = 5120


def _sc_mesh():
    return plsc.VectorSubcoreMesh(core_axis_name="c", subcore_axis_name="s")


@functools.partial(
    pl.kernel,
    out_type=jax.ShapeDtypeStruct((N_TILES, NPAD), jnp.float32),
    mesh=_sc_mesh(),
    compiler_params=pltpu.CompilerParams(needs_layout_passes=False),
    scratch_types=[
        pltpu.VMEM((CPT, EC), jnp.int32),
        pltpu.VMEM((NPAD,), jnp.float32),
    ],
)
def _sc_degree(edges_hbm, out_hbm, dst_v, hist_v):
    cid = lax.axis_index("c")
    sid = lax.axis_index("s")
    wid = sid * 2 + cid

    def zero_body(i, _):
        hist_v[pl.ds(i * 16, 16)] = jnp.zeros((16,), jnp.float32)
        return 0

    lax.fori_loop(0, NPAD // 16, zero_body, 0)

    pltpu.sync_copy(edges_hbm.at[1, pl.ds(wid * CPT, CPT)], dst_v)

    ones = jnp.full((16,), 1.0, jnp.float32)

    def body(r, _):
        for j in range(EC // 16):
            idx = dst_v[r, pl.ds(j * 16, 16)]
            plsc.addupdate_scatter(hist_v, [idx], ones)
        return 0

    lax.fori_loop(0, CPT, body, 0)
    pltpu.sync_copy(hist_v, out_hbm.at[wid])


@functools.partial(
    pl.kernel,
    out_type=jax.ShapeDtypeStruct((2, NPAD, D), jnp.float32),
    mesh=_sc_mesh(),
    compiler_params=pltpu.CompilerParams(needs_layout_passes=False),
    scratch_types=[
        pltpu.VMEM((2, 2, PHW, EC), jnp.int32),
        pltpu.VMEM((2, EC, D), jnp.float32),
        pltpu.VMEM_SHARED((NPAD, D), jnp.float32),
        pltpu.SemaphoreType.DMA,
        pltpu.SemaphoreType.DMA,
        pltpu.SemaphoreType.DMA,
    ],
)
def _sc_edge_pass(table_hbm, edges_hbm, out_hbm, idx_v, rows_v, acc_sh,
                  sem_a, sem_b, isem):
    cid = lax.axis_index("c")
    sid = lax.axis_index("s")
    wid = sid * 2 + cid

    def zero_body(i, _):
        for j in range(D // 16):
            rows_v[0, i, pl.ds(j * 16, 16)] = jnp.zeros((16,), jnp.float32)
        return 0

    lax.fori_loop(0, EC, zero_body, 0)
    row0 = sid * ROWS_PER_TILE
    for t in range(ROWS_PER_TILE // EC):
        pltpu.sync_copy(rows_v.at[0], acc_sh.at[pl.ds(row0 + t * EC, EC)])

    base = wid * CPT
    pltpu.sync_copy(edges_hbm.at[:, pl.ds(base, PHW)], idx_v.at[0])
    ipf = pltpu.async_copy(edges_hbm.at[:, pl.ds(base + PHW, PHW)],
                           idx_v.at[1], isem)

    plsc.subcore_barrier()

    def gather_pair(pp, cc, buf):
        a = pltpu.async_copy(table_hbm.at[idx_v.at[pp, 0, cc, pl.ds(0, EH)]],
                             rows_v.at[buf, pl.ds(0, EH)], sem_a)
        b = pltpu.async_copy(table_hbm.at[idx_v.at[pp, 0, cc, pl.ds(EH, EH)]],
                             rows_v.at[buf, pl.ds(EH, EH)], sem_b)
        return a, b

    cp = gather_pair(0, 0, 0)
    for g in range(CPT):
        p, c = divmod(g, PHW)
        for d in cp:
            d.wait()
        if g + 1 < CPT:
            p1, c1 = divmod(g + 1, PHW)
            if c1 == 0:
                ipf.wait()
            cp = gather_pair(p1 % 2, c1, (g + 1) % 2)
        pltpu.sync_copy(rows_v.at[g % 2], acc_sh.at[idx_v.at[p % 2, 1, c]],
                        add=True)
        if c == PHW - 1 and p + 2 < NPH:
            ipf = pltpu.async_copy(
                edges_hbm.at[:, pl.ds(base + (p + 2) * PHW, PHW)],
                idx_v.at[p % 2], isem)

    plsc.subcore_barrier()
    pltpu.sync_copy(acc_sh.at[pl.ds(row0, ROWS_PER_TILE)],
                    out_hbm.at[cid, pl.ds(row0, ROWS_PER_TILE)])


def _tc_layer1(x_pad, W1, degT):
    def body(x_ref, w_ref, deg_ref, t_ref, dinv_ref):
        deg = jnp.sum(deg_ref[...], axis=1, keepdims=True) + 1.0
        dinv = lax.rsqrt(jnp.maximum(deg, 1.0))
        t_ref[...] = jnp.dot(x_ref[...], w_ref[...],
                             preferred_element_type=jnp.float32) * dinv
        dinv_ref[...] = dinv

    return pl.pallas_call(
        body,
        grid=(NPAD // _R,),
        in_specs=[
            pl.BlockSpec((_R, D), lambda i: (i, 0)),
            pl.BlockSpec((D, D), lambda i: (0, 0)),
            pl.BlockSpec((_R, N_TILES), lambda i: (i, 0)),
        ],
        out_specs=[
            pl.BlockSpec((_R, D), lambda i: (i, 0)),
            pl.BlockSpec((_R, 1), lambda i: (i, 0)),
        ],
        out_shape=[
            jax.ShapeDtypeStruct((NPAD, D), jnp.float32),
            jax.ShapeDtypeStruct((NPAD, 1), jnp.float32),
        ],
    )(x_pad, W1, degT)


def _tc_mid(s1, table1, dinv, b1r, W2):
    def body(s_ref, t_ref, dinv_ref, b_ref, w_ref, o_ref):
        h = jax.nn.relu((s_ref[0] + s_ref[1] + t_ref[...]) * dinv_ref[...]
                        + b_ref[...])
        o_ref[...] = jnp.dot(h, w_ref[...],
                             preferred_element_type=jnp.float32) * dinv_ref[...]

    return pl.pallas_call(
        body,
        grid=(NPAD // _R,),
        in_specs=[
            pl.BlockSpec((2, _R, D), lambda i: (0, i, 0)),
            pl.BlockSpec((_R, D), lambda i: (i, 0)),
            pl.BlockSpec((_R, 1), lambda i: (i, 0)),
            pl.BlockSpec((1, D), lambda i: (0, 0)),
            pl.BlockSpec((D, D), lambda i: (0, 0)),
        ],
        out_specs=pl.BlockSpec((_R, D), lambda i: (i, 0)),
        out_shape=jax.ShapeDtypeStruct((NPAD, D), jnp.float32),
    )(s1, table1, dinv, b1r, W2)


def _tc_final(s2, table2, dinv, b2r, batch_row, hlr, std,
              Wf1a, w_hlr, w_std, bf1r, Wf2, bf2r):
    def body(s_ref, t_ref, dinv_ref, b_ref, batch_ref, hlr_ref, std_ref,
             wa_ref, wh_ref, ws_ref, bf1_ref, wf2_ref, bf2_ref,
             o_ref, sums, counts):
        i = pl.program_id(0)
        h2 = jax.nn.relu((s_ref[0] + s_ref[1] + t_ref[...]) * dinv_ref[...]
                         + b_ref[...])
        gids = lax.broadcasted_iota(jnp.int32, (N_GRAPHS, _R), 0)
        ind = (batch_ref[...] == gids).astype(jnp.float32)
        psum = jnp.dot(ind, h2, preferred_element_type=jnp.float32)
        pcnt = jnp.sum(ind, axis=1, keepdims=True)

        @pl.when(i == 0)
        def _():
            sums[...] = jnp.zeros_like(sums)
            counts[...] = jnp.zeros_like(counts)

        sums[...] += psum
        counts[...] += pcnt

        @pl.when(i == pl.num_programs(0) - 1)
        def _():
            pooled = sums[...] / jnp.maximum(counts[...], 1.0)
            hfc = jax.nn.relu(
                jnp.dot(pooled, wa_ref[...], preferred_element_type=jnp.float32)
                + hlr_ref[...] * wh_ref[...]
                + std_ref[...] * ws_ref[...]
                + bf1_ref[...])
            o_ref[...] = (jnp.dot(hfc, wf2_ref[...],
                                  preferred_element_type=jnp.float32)
                          + bf2_ref[...])

    return pl.pallas_call(
        body,
        grid=(NPAD // _R,),
        in_specs=[
            pl.BlockSpec((2, _R, D), lambda i: (0, i, 0)),
            pl.BlockSpec((_R, D), lambda i: (i, 0)),
            pl.BlockSpec((_R, 1), lambda i: (i, 0)),
            pl.BlockSpec((1, D), lambda i: (0, 0)),
            pl.BlockSpec((1, _R), lambda i: (0, i)),
            pl.BlockSpec((N_GRAPHS, 1), lambda i: (0, 0)),
            pl.BlockSpec((N_GRAPHS, 1), lambda i: (0, 0)),
            pl.BlockSpec((D, D), lambda i: (0, 0)),
            pl.BlockSpec((1, D), lambda i: (0, 0)),
            pl.BlockSpec((1, D), lambda i: (0, 0)),
            pl.BlockSpec((1, D), lambda i: (0, 0)),
            pl.BlockSpec((D, D), lambda i: (0, 0)),
            pl.BlockSpec((1, D), lambda i: (0, 0)),
        ],
        out_specs=pl.BlockSpec((N_GRAPHS, D), lambda i: (0, 0)),
        out_shape=jax.ShapeDtypeStruct((N_GRAPHS, D), jnp.float32),
        scratch_shapes=[
            pltpu.VMEM((N_GRAPHS, D), jnp.float32),
            pltpu.VMEM((N_GRAPHS, 1), jnp.float32),
        ],
    )(s2, table2, dinv, b2r, batch_row, hlr, std,
      Wf1a, w_hlr, w_std, bf1r, Wf2, bf2r)


def kernel(x, edge_index, batch, hlr, std, W1, b1, W2, b2, Wf1, bf1, Wf2, bf2):
    n_edges = edge_index.shape[1]
    n_extra = EPAD - n_edges
    pad_idx = N_NODES + (jnp.arange(n_extra, dtype=jnp.int32) % (NPAD - N_NODES))
    pad2 = jnp.broadcast_to(pad_idx, (2, n_extra))
    edges3d = jnp.concatenate([edge_index, pad2], axis=1).reshape(2, -1, EC)

    x_pad = jnp.pad(x, ((0, NPAD - N_NODES), (0, 0)))
    batch_row = jnp.pad(batch, (0, NPAD - N_NODES),
                        constant_values=N_GRAPHS).reshape(1, NPAD)

    b1r = b1.reshape(1, D)
    b2r = b2.reshape(1, D)
    bf1r = bf1.reshape(1, D)
    bf2r = bf2.reshape(1, D)
    Wf1a = Wf1[:D]
    w_hlr = Wf1[D:D + 1]
    w_std = Wf1[D + 1:D + 2]

    deg_parts = _sc_degree(edges3d)
    degT = jnp.transpose(deg_parts)

    table1, dinv = _tc_layer1(x_pad, W1, degT)
    s1 = _sc_edge_pass(table1, edges3d)
    table2 = _tc_mid(s1, table1, dinv, b1r, W2)
    s2 = _sc_edge_pass(table2, edges3d)
    return _tc_final(s2, table2, dinv, b2r, batch_row, hlr, std,
                     Wf1a, w_hlr, w_std, bf1r, Wf2, bf2r)
